# Initial kernel scaffold; baseline (speedup 1.0000x reference)
import jax
import jax.numpy as jnp
from jax.experimental import pallas as pl


def kernel(x, edge_index, Ws, bs, gammas, betas, linW, linb):
    raise NotImplementedError("write your pallas kernel here")


# speedup vs baseline: 6.9971x; 6.9971x over previous
import functools

import jax
import jax.numpy as jnp
from jax import lax
from jax.experimental import pallas as pl
from jax.experimental.pallas import tpu as pltpu
from jax.experimental.pallas import tpu_sc as plsc

F32 = jnp.float32
NC = 2
NS = 16
K_EDGE = 80
WSC = 128
BN = 1000


@functools.lru_cache(maxsize=None)
def _make_sc_agg(n_edges, n_pad, mode):
    ept = n_edges // NS if mode == "cols" else n_edges // (NC * NS)
    assert ept % K_EDGE == 0
    nchunks = ept // K_EDGE
    rpt = n_pad // NS
    mesh = plsc.VectorSubcoreMesh(core_axis_name="c", subcore_axis_name="s",
                                  num_cores=NC, num_subcores=NS)

    @functools.partial(
        pl.kernel,
        mesh=mesh,
        out_type=jax.ShapeDtypeStruct((NC, n_pad, WSC), F32),
        scratch_types=[
            pltpu.VMEM_SHARED((n_pad, WSC), F32),
            pltpu.VMEM((1, K_EDGE), jnp.int32),
            pltpu.VMEM((1, K_EDGE), jnp.int32),
            pltpu.VMEM((K_EDGE, WSC), F32),
        ],
    )
    def sc_agg(u_hbm, src_hbm, dst_hbm, zeros_hbm, out_hbm, acc, isrc, idst, rows):
        c = lax.axis_index("c")
        s = lax.axis_index("s")
        r0 = s * rpt
        pltpu.sync_copy(zeros_hbm.at[pl.ds(r0, rpt)], acc.at[pl.ds(r0, rpt)])
        if mode == "ones":
            pltpu.sync_copy(u_hbm, rows)
        plsc.subcore_barrier()
        if mode == "cols":
            base = s * ept
            gsrc = u_hbm.at[c]
        else:
            base = (c * NS + s) * ept
            gsrc = u_hbm

        def body(i, carry):
            off = base + i * K_EDGE
            pltpu.sync_copy(dst_hbm.at[pl.ds(off, K_EDGE)], idst.at[0])
            if mode != "ones":
                pltpu.sync_copy(src_hbm.at[pl.ds(off, K_EDGE)], isrc.at[0])
                pltpu.sync_copy(gsrc.at[isrc.at[0]], rows)
            pltpu.sync_copy(rows, acc.at[idst.at[0]], add=True)
            return carry

        lax.fori_loop(0, nchunks, body, 0)
        plsc.subcore_barrier()
        pltpu.sync_copy(acc.at[pl.ds(r0, rpt)], out_hbm.at[c].at[pl.ds(r0, rpt)])

    return sc_agg


def _full(shape):
    return pl.BlockSpec(shape, lambda i: (0,) * len(shape))


def _pad_cols(u, w):
    if u.shape[1] == w:
        return u
    return jnp.concatenate([u, jnp.zeros((u.shape[0], w - u.shape[1]), F32)], axis=1)


def _tc_prep(deg, xpad, n):
    def body(deg_ref, x_ref, dinv_ref, u_ref):
        d = deg_ref[0, :, 0:1] + deg_ref[1, :, 0:1] + 1.0
        dv = lax.rsqrt(d)
        dinv_ref[...] = dv
        u_ref[...] = x_ref[...] * dv

    return pl.pallas_call(
        body,
        grid=(n // BN,),
        in_specs=[
            pl.BlockSpec((2, BN, WSC), lambda i: (0, i, 0)),
            pl.BlockSpec((BN, WSC), lambda i: (i, 0)),
        ],
        out_specs=[
            pl.BlockSpec((BN, 1), lambda i: (i, 0)),
            pl.BlockSpec((BN, WSC), lambda i: (i, 0)),
        ],
        out_shape=[
            jax.ShapeDtypeStruct((n, 1), F32),
            jax.ShapeDtypeStruct((n, WSC), F32),
        ],
    )(deg, xpad)


def _tc_post(agg, u, dinv, wmat, b, n, mode, matmul, hout):
    nb = n // BN

    def body(agg_ref, u_ref, dinv_ref, *rest):
        if matmul:
            w_ref, b_ref, z_ref, st_ref = rest
        else:
            b_ref, z_ref, st_ref = rest
        if mode == "cols":
            a = jnp.concatenate([agg_ref[0], agg_ref[1]], axis=1)
            uu = jnp.concatenate([u_ref[0], u_ref[1]], axis=1)
        else:
            a = agg_ref[0] + agg_ref[1]
            uu = u_ref[...]
        m = dinv_ref[...] * (a + uu)
        if matmul:
            z = lax.dot_general(m, w_ref[...], (((1,), (1,)), ((), ())),
                                preferred_element_type=F32,
                                precision=lax.Precision.HIGHEST) + b_ref[...]
        else:
            z = m[:, :hout] + b_ref[...]
        z_ref[...] = z

        @pl.when(pl.program_id(0) == 0)
        def _():
            st_ref[...] = jnp.zeros_like(st_ref)

        st_ref[0:1, :] += jnp.sum(z, axis=0, keepdims=True)
        st_ref[1:2, :] += jnp.sum(z * z, axis=0, keepdims=True)

    u_spec = (pl.BlockSpec((2, BN, WSC), lambda i: (0, i, 0)) if mode == "cols"
              else pl.BlockSpec((BN, WSC), lambda i: (i, 0)))
    in_specs = [
        pl.BlockSpec((2, BN, WSC), lambda i: (0, i, 0)),
        u_spec,
        pl.BlockSpec((BN, 1), lambda i: (i, 0)),
    ]
    args = [agg, u, dinv]
    if matmul:
        in_specs.append(_full(wmat.shape))
        args.append(wmat)
    in_specs.append(_full((1, hout)))
    args.append(b.reshape(1, hout))
    return pl.pallas_call(
        body,
        grid=(nb,),
        in_specs=in_specs,
        out_specs=[
            pl.BlockSpec((BN, hout), lambda i: (i, 0)),
            _full((2, hout)),
        ],
        out_shape=[
            jax.ShapeDtypeStruct((n, hout), F32),
            jax.ShapeDtypeStruct((2, hout), F32),
        ],
    )(*args)


def _tc_pre(z, st, g, bt, dinv, wmat, n, out_mode, matmul):
    hin = z.shape[1]
    wout = wmat.shape[0] if matmul else hin
    inv_n = 1.0 / n

    def body(z_ref, st_ref, g_ref, bt_ref, dinv_ref, *rest):
        if matmul:
            w_ref, u_ref = rest
        else:
            (u_ref,) = rest
        mean = st_ref[0:1, :] * inv_n
        var = st_ref[1:2, :] * inv_n - mean * mean
        rstd = lax.rsqrt(var + 1e-5)
        h = (z_ref[...] - mean) * rstd * g_ref[...] + bt_ref[...]
        h = jnp.where(h >= 0, h, 0.01 * h)
        if matmul:
            u = lax.dot_general(h, w_ref[...], (((1,), (1,)), ((), ())),
                                preferred_element_type=F32,
                                precision=lax.Precision.HIGHEST) * dinv_ref[...]
        else:
            u = h * dinv_ref[...]
        if out_mode == "cols":
            u_ref[0, :, :] = u[:, :WSC]
            u_ref[1, :, :] = u[:, WSC:]
        else:
            u_ref[...] = _pad_cols(u, WSC)

    in_specs = [
        pl.BlockSpec((BN, hin), lambda i: (i, 0)),
        _full((2, hin)),
        _full((1, hin)),
        _full((1, hin)),
        pl.BlockSpec((BN, 1), lambda i: (i, 0)),
    ]
    args = [z, st, g.reshape(1, hin), bt.reshape(1, hin), dinv]
    if matmul:
        in_specs.append(_full(wmat.shape))
        args.append(wmat)
    if out_mode == "cols":
        out_spec = pl.BlockSpec((2, BN, WSC), lambda i: (0, i, 0))
        out_shape = jax.ShapeDtypeStruct((2, n, WSC), F32)
    else:
        out_spec = pl.BlockSpec((BN, WSC), lambda i: (i, 0))
        out_shape = jax.ShapeDtypeStruct((n, WSC), F32)
    return pl.pallas_call(
        body,
        grid=(n // BN,),
        in_specs=in_specs,
        out_specs=out_spec,
        out_shape=out_shape,
    )(*args)


def _tc_head(z, st, g, bt, linw, linb, n):
    hin = z.shape[1]
    hout = linw.shape[0]
    inv_n = 1.0 / n

    def body(z_ref, st_ref, g_ref, bt_ref, w_ref, b_ref, o_ref):
        mean = st_ref[0:1, :] * inv_n
        var = st_ref[1:2, :] * inv_n - mean * mean
        rstd = lax.rsqrt(var + 1e-5)
        h = (z_ref[...] - mean) * rstd * g_ref[...] + bt_ref[...]
        h = jnp.where(h >= 0, h, 0.01 * h)
        o = lax.dot_general(h, w_ref[...], (((1,), (1,)), ((), ())),
                            preferred_element_type=F32,
                                precision=lax.Precision.HIGHEST) + b_ref[...]
        t = jnp.tanh(o)
        nrm = jnp.sqrt(jnp.sum(t * t, axis=1, keepdims=True))
        o_ref[...] = t * (1.0 / (nrm + 1e-12))

    return pl.pallas_call(
        body,
        grid=(n // BN,),
        in_specs=[
            pl.BlockSpec((BN, hin), lambda i: (i, 0)),
            _full((2, hin)),
            _full((1, hin)),
            _full((1, hin)),
            _full(linw.shape),
            _full((1, hout)),
        ],
        out_specs=pl.BlockSpec((BN, hout), lambda i: (i, 0)),
        out_shape=jax.ShapeDtypeStruct((n, hout), F32),
    )(z, st, g.reshape(1, hin), bt.reshape(1, hin), linw, linb.reshape(1, hout))


def kernel(x, edge_index, Ws, bs, gammas, betas, linW, linb):
    n = x.shape[0]
    n_edges = edge_index.shape[1]
    n_pad = ((n + NS * 8 - 1) // (NS * 8)) * (NS * 8)
    src = edge_index[0]
    dst = edge_index[1]

    nlayers = len(Ws)
    h_in = [w.shape[1] for w in Ws]
    h_out = [w.shape[0] for w in Ws]
    flavors = ["B" if h_in[i] < h_out[i] else "A" for i in range(nlayers)]
    waggs = [h_in[i] if flavors[i] == "B" else h_out[i] for i in range(nlayers)]
    modes = ["cols" if w > WSC else "edges" for w in waggs]

    xpad = jnp.pad(x, ((0, 0), (0, WSC - h_in[0])))
    wpads = [jnp.pad(w, ((0, 0), (0, WSC - w.shape[1]))) if
             (flavors[i] == "B" and w.shape[1] < WSC) else w
             for i, w in enumerate(Ws)]

    zeros = jnp.zeros((n_pad, WSC), F32)
    ones_rows = jnp.ones((K_EDGE, WSC), F32)

    deg = _make_sc_agg(n_edges, n_pad, "ones")(ones_rows, src, dst, zeros)
    dinv, u = _tc_prep(deg, xpad, n)

    z, st = None, None
    for i in range(nlayers):
        mode_i = modes[i]
        if i > 0:
            wm = Ws[i] if flavors[i] == "A" else None
            u = _tc_pre(z, st, gammas[i - 1], betas[i - 1], dinv, wm, n,
                        mode_i, flavors[i] == "A")
        agg = _make_sc_agg(n_edges, n_pad, mode_i)(u, src, dst, zeros)
        wm_post = wpads[i] if flavors[i] == "B" else None
        z, st = _tc_post(agg, u, dinv, wm_post, bs[i], n, mode_i,
                         flavors[i] == "B", h_out[i])

    return _tc_head(z, st, gammas[-1], betas[-1], linW, linb, n)
